# baseline (speedup 1.0000x reference)
import functools

import jax
import jax.numpy as jnp
from jax import lax
from jax.experimental import pallas as pl
from jax.experimental.pallas import tpu as pltpu
from jax.experimental.pallas import tpu_sc as plsc

D = 128
B = 16384
T = 20

NC = 2
NS = 16
NW = NC * NS
NPW = B // NW
ROWS_W = NPW * T

IDX_COLS = 80
IDX_ROWS = ROWS_W // IDX_COLS
CHUNK_NODES = 8
CHUNK_ROWS = CHUNK_NODES * T
IDX_PER_CHUNK = CHUNK_ROWS // IDX_COLS
N_CHUNKS = NPW // CHUNK_NODES
N_SUPER = N_CHUNKS // 2


def _node_embed_sc(buckets, tok):
    mesh = plsc.VectorSubcoreMesh(core_axis_name="c", subcore_axis_name="s")

    @functools.partial(
        pl.kernel,
        mesh=mesh,
        out_type=jax.ShapeDtypeStruct((B, D), jnp.float32),
        scratch_types=[
            pltpu.VMEM((IDX_ROWS, IDX_COLS), jnp.int32),
            pltpu.VMEM((CHUNK_ROWS, D), jnp.float32),
            pltpu.VMEM((CHUNK_ROWS, D), jnp.float32),
            pltpu.VMEM((NPW, D), jnp.float32),
            pltpu.SemaphoreType.DMA,
            pltpu.SemaphoreType.DMA,
            pltpu.SemaphoreType.DMA,
        ],
    )
    def k(table_hbm, tok_hbm, out_hbm, idx_v, rows0, rows1, out_v, sem0, sem1, semo):
        i32 = jnp.int32
        wid = lax.axis_index("s") * i32(NC) + lax.axis_index("c")
        pltpu.sync_copy(tok_hbm.at[wid], idx_v)

        def fire(c, buf, sem):
            for j in range(IDX_PER_CHUNK):
                pltpu.async_copy(
                    table_hbm.at[idx_v.at[c * i32(IDX_PER_CHUNK) + i32(j)]],
                    buf.at[pl.ds(j * IDX_COLS, IDX_COLS)],
                    sem,
                )

        def drain(c, buf, sem):
            for j in range(IDX_PER_CHUNK):
                pltpu.make_async_copy(
                    table_hbm.at[idx_v.at[c * i32(IDX_PER_CHUNK) + i32(j)]],
                    buf.at[pl.ds(j * IDX_COLS, IDX_COLS)],
                    sem,
                ).wait()

        def compute(c, buf):
            out_base = c * i32(CHUNK_NODES)
            for n in range(CHUNK_NODES):
                base = n * T
                for d in range(D // 16):
                    sl = pl.ds(d * 16, 16)
                    acc = buf[base, sl]
                    for t in range(1, T):
                        acc = acc + buf[base + t, sl]
                    out_v[out_base + i32(n), sl] = acc

        fire(i32(0), rows0, sem0)

        def g_body(g, carry):
            c0 = g * i32(2)
            c1 = c0 + i32(1)
            fire(c1, rows1, sem1)
            drain(c0, rows0, sem0)
            compute(c0, rows0)

            @pl.when(g < i32(N_SUPER - 1))
            def _():
                fire(c0 + i32(2), rows0, sem0)

            drain(c1, rows1, sem1)
            compute(c1, rows1)
            return carry

        lax.fori_loop(0, i32(N_SUPER), g_body, i32(0))

        pltpu.async_copy(out_v, out_hbm.at[pl.ds(wid * i32(NPW), NPW)], semo).wait()

    return k(buckets, tok)


def kernel(buckets, node_ids, token_ids):
    del node_ids
    tok = token_ids.astype(jnp.int32).reshape(NW, IDX_ROWS, IDX_COLS)
    return _node_embed_sc(buckets, tok)


# speedup vs baseline: 2.3023x; 2.3023x over previous
import functools

import jax
import jax.numpy as jnp
from jax import lax
from jax.experimental import pallas as pl
from jax.experimental.pallas import tpu as pltpu
from jax.experimental.pallas import tpu_sc as plsc

D = 128
B = 16384
T = 20

NC = 2
NS = 16
NW = NC * NS
NPW = B // NW
ROWS_W = NPW * T

IDX_COLS = 80
IDX_ROWS = ROWS_W // IDX_COLS
CHUNK_NODES = 8
CHUNK_ROWS = CHUNK_NODES * T
IDX_PER_CHUNK = CHUNK_ROWS // IDX_COLS
N_CHUNKS = NPW // CHUNK_NODES
N_SUPER = N_CHUNKS // 2


def _node_embed_sc(buckets, tok):
    mesh = plsc.VectorSubcoreMesh(core_axis_name="c", subcore_axis_name="s")

    @functools.partial(
        pl.kernel,
        mesh=mesh,
        out_type=jax.ShapeDtypeStruct((B, D), jnp.float32),
        scratch_types=[
            pltpu.VMEM((IDX_ROWS, IDX_COLS), jnp.int32),
            pltpu.VMEM((CHUNK_ROWS, D), jnp.float32),
            pltpu.VMEM((CHUNK_ROWS, D), jnp.float32),
            pltpu.VMEM((NPW, D), jnp.float32),
            pltpu.SemaphoreType.DMA,
            pltpu.SemaphoreType.DMA,
            pltpu.SemaphoreType.DMA,
        ],
    )
    def k(table_hbm, tok_hbm, out_hbm, idx_v, rows0, rows1, out_v, sem0, sem1, semo):
        i32 = jnp.int32
        wid = lax.axis_index("s") * i32(NC) + lax.axis_index("c")
        pltpu.sync_copy(tok_hbm.at[wid], idx_v)

        def fire(c, buf, sem):
            for j in range(IDX_PER_CHUNK):
                pltpu.async_copy(
                    table_hbm.at[idx_v.at[c * i32(IDX_PER_CHUNK) + i32(j)]],
                    buf.at[pl.ds(j * IDX_COLS, IDX_COLS)],
                    sem,
                )

        def drain(c, buf, sem):
            for j in range(IDX_PER_CHUNK):
                pltpu.make_async_copy(
                    table_hbm.at[idx_v.at[c * i32(IDX_PER_CHUNK) + i32(j)]],
                    buf.at[pl.ds(j * IDX_COLS, IDX_COLS)],
                    sem,
                ).wait()

        def compute(c, buf):
            out_base = c * i32(CHUNK_NODES)

            def node_body(n, c2):
                base = n * i32(T)
                for d in range(D // 16):
                    sl = pl.ds(d * 16, 16)
                    acc = buf[base, sl]
                    for t in range(1, T):
                        acc = acc + buf[base + i32(t), sl]
                    out_v[out_base + n, sl] = acc
                return c2

            lax.fori_loop(0, i32(CHUNK_NODES), node_body, i32(0))

        fire(i32(0), rows0, sem0)

        def g_body(g, carry):
            c0 = g * i32(2)
            c1 = c0 + i32(1)
            fire(c1, rows1, sem1)
            drain(c0, rows0, sem0)
            compute(c0, rows0)

            @pl.when(g < i32(N_SUPER - 1))
            def _():
                fire(c0 + i32(2), rows0, sem0)

            drain(c1, rows1, sem1)
            compute(c1, rows1)
            return carry

        lax.fori_loop(0, i32(N_SUPER), g_body, i32(0))

        pltpu.async_copy(out_v, out_hbm.at[pl.ds(wid * i32(NPW), NPW)], semo).wait()

    return k(buckets, tok)


def kernel(buckets, node_ids, token_ids):
    del node_ids
    tok = token_ids.astype(jnp.int32).reshape(NW, IDX_ROWS, IDX_COLS)
    return _node_embed_sc(buckets, tok)


# speedup vs baseline: 2.8290x; 1.2288x over previous
import functools

import jax
import jax.numpy as jnp
from jax import lax
from jax.experimental import pallas as pl
from jax.experimental.pallas import tpu as pltpu
from jax.experimental.pallas import tpu_sc as plsc

D = 128
B = 16384
T = 20

NC = 2
NS = 16
NW = NC * NS
NPW = B // NW
ROWS_W = NPW * T

IDX_COLS = 80
IDX_ROWS = ROWS_W // IDX_COLS
CHUNK_NODES = 8
CHUNK_ROWS = CHUNK_NODES * T
IDX_PER_CHUNK = CHUNK_ROWS // IDX_COLS
N_CHUNKS = NPW // CHUNK_NODES
N_SUPER = N_CHUNKS // 2


def _node_embed_sc(buckets, tok):
    mesh = plsc.VectorSubcoreMesh(core_axis_name="c", subcore_axis_name="s")

    @functools.partial(
        pl.kernel,
        mesh=mesh,
        out_type=jax.ShapeDtypeStruct((B, D), jnp.float32),
        scratch_types=[
            pltpu.VMEM((IDX_ROWS, IDX_COLS), jnp.int32),
            pltpu.VMEM((CHUNK_ROWS, D), jnp.float32),
            pltpu.VMEM((CHUNK_ROWS, D), jnp.float32),
            pltpu.VMEM((NPW, D), jnp.float32),
            pltpu.SemaphoreType.DMA,
            pltpu.SemaphoreType.DMA,
            pltpu.SemaphoreType.DMA,
        ],
    )
    def k(table_hbm, tok_hbm, out_hbm, idx_v, rows0, rows1, out_v, sem0, sem1, semo):
        i32 = jnp.int32
        wid = lax.axis_index("s") * i32(NC) + lax.axis_index("c")
        pltpu.sync_copy(tok_hbm.at[wid], idx_v)

        def fire(c, buf, sem):
            for j in range(IDX_PER_CHUNK):
                pltpu.async_copy(
                    table_hbm.at[idx_v.at[c * i32(IDX_PER_CHUNK) + i32(j)]],
                    buf.at[pl.ds(j * IDX_COLS, IDX_COLS)],
                    sem,
                )

        def drain(c, buf, sem):
            for j in range(IDX_PER_CHUNK):
                pltpu.make_async_copy(
                    table_hbm.at[idx_v.at[c * i32(IDX_PER_CHUNK) + i32(j)]],
                    buf.at[pl.ds(j * IDX_COLS, IDX_COLS)],
                    sem,
                ).wait()

        def compute(c, buf):
            out_base = c * i32(CHUNK_NODES)

            def node_body(n, c2):
                base = n * i32(T)
                accs = [buf[base, pl.ds(d * 16, 16)] for d in range(D // 16)]
                for t in range(1, T):
                    row = base + i32(t)
                    for d in range(D // 16):
                        accs[d] = accs[d] + buf[row, pl.ds(d * 16, 16)]
                for d in range(D // 16):
                    out_v[out_base + n, pl.ds(d * 16, 16)] = accs[d]
                return c2

            lax.fori_loop(0, i32(CHUNK_NODES), node_body, i32(0))

        fire(i32(0), rows0, sem0)

        def g_body(g, carry):
            c0 = g * i32(2)
            c1 = c0 + i32(1)
            fire(c1, rows1, sem1)
            drain(c0, rows0, sem0)
            compute(c0, rows0)

            @pl.when(g < i32(N_SUPER - 1))
            def _():
                fire(c0 + i32(2), rows0, sem0)

            drain(c1, rows1, sem1)
            compute(c1, rows1)
            return carry

        lax.fori_loop(0, i32(N_SUPER), g_body, i32(0))

        pltpu.async_copy(out_v, out_hbm.at[pl.ds(wid * i32(NPW), NPW)], semo).wait()

    return k(buckets, tok)


def kernel(buckets, node_ids, token_ids):
    del node_ids
    tok = token_ids.astype(jnp.int32).reshape(NW, IDX_ROWS, IDX_COLS)
    return _node_embed_sc(buckets, tok)


# speedup vs baseline: 3.3905x; 1.1985x over previous
import functools

import jax
import jax.numpy as jnp
from jax import lax
from jax.experimental import pallas as pl
from jax.experimental.pallas import tpu as pltpu
from jax.experimental.pallas import tpu_sc as plsc

D = 128
B = 16384
T = 20

NC = 2
NS = 16
NW = NC * NS
NPW = B // NW
ROWS_W = NPW * T

CHUNK_NODES = 4
CHUNK_ROWS = CHUNK_NODES * T
IDX_ROWS = ROWS_W // CHUNK_ROWS
N_CHUNKS = NPW // CHUNK_NODES
RING = 4
N_SUPER = N_CHUNKS // RING


def _node_embed_sc(buckets, tok):
    mesh = plsc.VectorSubcoreMesh(core_axis_name="c", subcore_axis_name="s")

    @functools.partial(
        pl.kernel,
        mesh=mesh,
        out_type=jax.ShapeDtypeStruct((B, D), jnp.float32),
        scratch_types=[
            pltpu.VMEM((IDX_ROWS, CHUNK_ROWS), jnp.int32),
            pltpu.VMEM((RING, CHUNK_ROWS, D), jnp.float32),
            pltpu.VMEM((NPW, D), jnp.float32),
            pltpu.SemaphoreType.DMA,
            pltpu.SemaphoreType.DMA,
            pltpu.SemaphoreType.DMA,
            pltpu.SemaphoreType.DMA,
            pltpu.SemaphoreType.DMA,
        ],
    )
    def k(table_hbm, tok_hbm, out_hbm, idx_v, rows_v, out_v,
          sem0, sem1, sem2, sem3, semo):
        i32 = jnp.int32
        sems = [sem0, sem1, sem2, sem3]
        wid = lax.axis_index("s") * i32(NC) + lax.axis_index("c")
        pltpu.sync_copy(tok_hbm.at[wid], idx_v)

        def fire(c, b):
            pltpu.async_copy(
                table_hbm.at[idx_v.at[c]],
                rows_v.at[jnp.int32(b)],
                sems[b],
            )

        def drain(c, b):
            pltpu.make_async_copy(
                table_hbm.at[idx_v.at[c]],
                rows_v.at[jnp.int32(b)],
                sems[b],
            ).wait()

        def compute(c, b):
            buf = rows_v.at[jnp.int32(b)]
            out_base = c * i32(CHUNK_NODES)

            def node_body(n, c2):
                base = n * i32(T)
                accs = [buf[base, pl.ds(d * 16, 16)] for d in range(D // 16)]
                for t in range(1, T):
                    row = base + i32(t)
                    for d in range(D // 16):
                        accs[d] = accs[d] + buf[row, pl.ds(d * 16, 16)]
                for d in range(D // 16):
                    out_v[out_base + n, pl.ds(d * 16, 16)] = accs[d]
                return c2

            lax.fori_loop(0, i32(CHUNK_NODES), node_body, i32(0))

        for b in range(RING - 1):
            fire(i32(b), b)

        def g_body(g, carry):
            c0 = g * i32(RING)
            for b in range(RING):
                c = c0 + i32(b)
                nxt = c + i32(RING - 1)

                @pl.when(nxt < i32(N_CHUNKS))
                def _():
                    fire(nxt, (b + RING - 1) % RING)

                drain(c, b)
                compute(c, b)
            return carry

        lax.fori_loop(0, i32(N_SUPER), g_body, i32(0))

        pltpu.async_copy(out_v, out_hbm.at[pl.ds(wid * i32(NPW), NPW)], semo).wait()

    return k(buckets, tok)


def kernel(buckets, node_ids, token_ids):
    del node_ids
    tok = token_ids.astype(jnp.int32).reshape(NW, IDX_ROWS, CHUNK_ROWS)
    return _node_embed_sc(buckets, tok)
